# baseline (speedup 1.0000x reference)
import functools

import jax
import jax.numpy as jnp
from jax import lax
from jax.experimental import pallas as pl
from jax.experimental.pallas import tpu as pltpu
from jax.experimental.pallas import tpu_sc as plsc

F = 64
GR = 2 * F
LANES = 16
NCORES = 2
NSUB = 16
NW = NCORES * NSUB
B = 16384
ROWS = 128
COLS = 128
RPW = ROWS // NW
BPW = RPW * COLS


def _sc_body(user_r, pos_r, neg_r, uw_r, iw_r, out_r,
             iu_v, ip_v, in_v, u0, p0, n0, u1, p1, n1, sem):
    wid = lax.axis_index("s") * NCORES + lax.axis_index("c")
    row0 = wid * RPW
    pltpu.sync_copy(user_r.at[pl.ds(row0, RPW)], iu_v)
    pltpu.sync_copy(pos_r.at[pl.ds(row0, RPW)], ip_v)
    pltpu.sync_copy(neg_r.at[pl.ds(row0, RPW)], in_v)
    for iv in (iu_v, ip_v, in_v):
        for j in range(RPW):
            for c in range(COLS // LANES):
                cs = pl.ds(c * LANES, LANES)
                iv[j, cs] = lax.shift_right_logical(iv[j, cs], 1)

    bufs = ((u0, p0, n0), (u1, p1, n1))

    def start(j):
        bu, bp, bn = bufs[j % 2]
        return (pltpu.async_copy(uw_r.at[iu_v.at[j]], bu, sem),
                pltpu.async_copy(iw_r.at[ip_v.at[j]], bp, sem),
                pltpu.async_copy(iw_r.at[in_v.at[j]], bn, sem))

    base = wid * BPW
    pend = start(0)
    for j in range(RPW):
        for c in pend:
            c.wait()
        if j + 1 < RPW:
            nxt = start(j + 1)
        bu, bp, bn = bufs[j % 2]
        dst = pl.ds(base + j * COLS, COLS)
        pltpu.sync_copy(bu, out_r.at[0, dst])
        pltpu.sync_copy(bp, out_r.at[1, dst])
        pltpu.sync_copy(bn, out_r.at[2, dst])
        if j + 1 < RPW:
            pend = nxt


def _tc_body(r_ref, iu_ref, ip_ref, in_ref, o_ref):
    def half(rows, idx):
        par = (idx & 1) == 1
        return jnp.where(par, rows[:, F:], rows[:, :F])

    us = half(r_ref[0], iu_ref[...])
    ps = half(r_ref[1], ip_ref[...])
    ns = half(r_ref[2], in_ref[...])
    y = -jnp.sum(us * (ps - ns), axis=1)
    part = jnp.sum(jnp.maximum(y, 0.0) + jnp.log1p(jnp.exp(-jnp.abs(y))))

    @pl.when(pl.program_id(0) == 0)
    def _():
        o_ref[0, 0] = 0.0

    o_ref[0, 0] += part


@jax.jit
def kernel(user, pos_i, neg_j, users_weight, items_weight):
    user2 = user.reshape(ROWS, COLS)
    pos2 = pos_i.reshape(ROWS, COLS)
    neg2 = neg_j.reshape(ROWS, COLS)
    uw2 = users_weight.reshape(-1, GR)
    iw2 = items_weight.reshape(-1, GR)
    mesh = plsc.VectorSubcoreMesh(core_axis_name="c", subcore_axis_name="s")
    sc = functools.partial(
        pl.kernel,
        mesh=mesh,
        out_type=jax.ShapeDtypeStruct((3, B, GR), jnp.float32),
        scratch_types=[
            pltpu.VMEM((RPW, COLS), jnp.int32),
            pltpu.VMEM((RPW, COLS), jnp.int32),
            pltpu.VMEM((RPW, COLS), jnp.int32),
            pltpu.VMEM((COLS, GR), jnp.float32),
            pltpu.VMEM((COLS, GR), jnp.float32),
            pltpu.VMEM((COLS, GR), jnp.float32),
            pltpu.VMEM((COLS, GR), jnp.float32),
            pltpu.VMEM((COLS, GR), jnp.float32),
            pltpu.VMEM((COLS, GR), jnp.float32),
            pltpu.SemaphoreType.DMA,
        ],
    )(_sc_body)
    rows = sc(user2, pos2, neg2, uw2, iw2)

    grid = 8
    bb = B // grid
    loss = pl.pallas_call(
        _tc_body,
        grid=(grid,),
        in_specs=[
            pl.BlockSpec((3, bb, GR), lambda i: (0, i, 0)),
            pl.BlockSpec((bb, 1), lambda i: (i, 0)),
            pl.BlockSpec((bb, 1), lambda i: (i, 0)),
            pl.BlockSpec((bb, 1), lambda i: (i, 0)),
        ],
        out_specs=pl.BlockSpec(memory_space=pltpu.SMEM),
        out_shape=jax.ShapeDtypeStruct((1, 1), jnp.float32),
    )(rows, user.reshape(B, 1), pos_i.reshape(B, 1), neg_j.reshape(B, 1))
    return loss[0, 0]


# speedup vs baseline: 1.0280x; 1.0280x over previous
import functools

import jax
import jax.numpy as jnp
from jax import lax
from jax.experimental import pallas as pl
from jax.experimental.pallas import tpu as pltpu
from jax.experimental.pallas import tpu_sc as plsc

F = 64
LANES = 16
NCORES = 2
NSUB = 16
NW = NCORES * NSUB
B = 16384
ROWS = 128
COLS = 128
RPW = ROWS // NW
BPW = RPW * COLS
GROUP = 16


def _sc_body(user_r, pos_r, neg_r, uw_r, iw_r, out_r,
             iu_v, ip_v, in_v, *rest):
    bufs = rest[:3 * RPW]
    accs = rest[3 * RPW]
    sem = rest[3 * RPW + 1]
    wid = lax.axis_index("s") * NCORES + lax.axis_index("c")
    row0 = wid * RPW
    pltpu.sync_copy(user_r.at[pl.ds(row0, RPW)], iu_v)
    pltpu.sync_copy(pos_r.at[pl.ds(row0, RPW)], ip_v)
    pltpu.sync_copy(neg_r.at[pl.ds(row0, RPW)], in_v)

    copies = []
    for j in range(RPW):
        bu, bp, bn = bufs[3 * j:3 * j + 3]
        copies.append((pltpu.async_copy(uw_r.at[iu_v.at[j]], bu, sem),
                       pltpu.async_copy(iw_r.at[ip_v.at[j]], bp, sem),
                       pltpu.async_copy(iw_r.at[in_v.at[j]], bn, sem)))

    base = wid * BPW
    for j in range(RPW):
        for c in copies[j]:
            c.wait()
        bu, bp, bn = bufs[3 * j:3 * j + 3]

        def group(g, carry):
            r0 = g * GROUP
            for rr in range(GROUP):
                r = r0 + rr
                acc = None
                for c4 in range(F // LANES):
                    sl = pl.ds(c4 * LANES, LANES)
                    prod = bu[r, sl] * (bp[r, sl] - bn[r, sl])
                    acc = prod if acc is None else acc + prod
                accs[r] = acc
            return carry

        lax.fori_loop(0, COLS // GROUP, group, 0)
        pltpu.sync_copy(accs, out_r.at[pl.ds(base + j * COLS, COLS)])


def _tc_body(a_ref, o_ref):
    x = a_ref[...]
    tot = jnp.zeros((), jnp.float32)
    for k in range(8):
        d = jnp.sum(x[:, k * LANES:(k + 1) * LANES], axis=1, keepdims=True)
        y = -d
        tot += jnp.sum(jnp.maximum(y, 0.0) + jnp.log1p(jnp.exp(-jnp.abs(y))))
    o_ref[0, 0] = tot


@jax.jit
def kernel(user, pos_i, neg_j, users_weight, items_weight):
    user2 = user.reshape(ROWS, COLS)
    pos2 = pos_i.reshape(ROWS, COLS)
    neg2 = neg_j.reshape(ROWS, COLS)
    mesh = plsc.VectorSubcoreMesh(core_axis_name="c", subcore_axis_name="s")
    sc = functools.partial(
        pl.kernel,
        mesh=mesh,
        compiler_params=pltpu.CompilerParams(use_tc_tiling_on_sc=False),
        out_type=jax.ShapeDtypeStruct((B, LANES), jnp.float32),
        scratch_types=(
            [pltpu.VMEM((RPW, COLS), jnp.int32)] * 3
            + [pltpu.VMEM((COLS, F), jnp.float32)] * (3 * RPW)
            + [pltpu.VMEM((COLS, LANES), jnp.float32),
               pltpu.SemaphoreType.DMA]
        ),
    )(_sc_body)
    accs = sc(user2, pos2, neg2, users_weight, items_weight)

    loss = pl.pallas_call(
        _tc_body,
        out_specs=pl.BlockSpec(memory_space=pltpu.SMEM),
        out_shape=jax.ShapeDtypeStruct((1, 1), jnp.float32),
    )(accs.reshape(B * LANES // COLS, COLS))
    return loss[0, 0]


# speedup vs baseline: 1.6500x; 1.6050x over previous
import functools

import jax
import jax.numpy as jnp
from jax import lax
from jax.experimental import pallas as pl
from jax.experimental.pallas import tpu as pltpu
from jax.experimental.pallas import tpu_sc as plsc

F = 64
GR = 2 * F
LANES = 16
NCORES = 2
NSUB = 16
NW = NCORES * NSUB
B = 16384
N_ROWS = 1000000
HALF = N_ROWS // 2
COLS = 128
RPW = 4
BPW = RPW * COLS
CB = 4096
TGRID = (N_ROWS + CB - 1) // CB
GRAN = TGRID * (CB // 2)


def _tr_body(x_ref, o_ref):
    eye = (lax.broadcasted_iota(jnp.int32, (F, F), 0)
           == lax.broadcasted_iota(jnp.int32, (F, F), 1)).astype(jnp.float32)
    dn = (((0,), (0,)), ((), ()))
    t = lax.dot_general(x_ref[...], eye, dn,
                        preferred_element_type=jnp.float32)
    o_ref[:, 0:F] = t[0:CB // 2]
    o_ref[:, F:GR] = t[CB // 2:CB]


def _transpose_table(tw):
    return pl.pallas_call(
        _tr_body,
        grid=(TGRID,),
        in_specs=[pl.BlockSpec((F, CB), lambda k: (0, k))],
        out_specs=pl.BlockSpec((CB // 2, GR), lambda k: (k, 0)),
        out_shape=jax.ShapeDtypeStruct((GRAN, GR), jnp.float32),
    )(tw)


def _sc_body(user_r, pos_r, neg_r, tu_r, ti_r, out_r,
             iu_v, ip_v, in_v, ou_v, op_v, on_v,
             u0, p0, n0, u1, p1, n1, accs, sem):
    wid = lax.axis_index("s") * NCORES + lax.axis_index("c")
    base = wid * BPW
    pltpu.sync_copy(user_r.at[pl.ds(base, BPW)], iu_v)
    pltpu.sync_copy(pos_r.at[pl.ds(base, BPW)], ip_v)
    pltpu.sync_copy(neg_r.at[pl.ds(base, BPW)], in_v)

    for iv, ov in ((iu_v, ou_v), (ip_v, op_v), (in_v, on_v)):
        for c in range(BPW // LANES):
            sl = pl.ds(c * LANES, LANES)
            v = iv[sl]
            blk = lax.shift_left(lax.shift_right_logical(v, 12), 11)
            iv[sl] = blk + (v & 2047)
            ov[sl] = (lax.shift_right_logical(v, 11) & 1) * F

    bufs = ((u0, p0, n0), (u1, p1, n1))

    def start(j):
        bu, bp, bn = bufs[j % 2]
        sl = pl.ds(j * COLS, COLS)
        return (pltpu.async_copy(tu_r.at[iu_v.at[sl]], bu, sem),
                pltpu.async_copy(ti_r.at[ip_v.at[sl]], bp, sem),
                pltpu.async_copy(ti_r.at[in_v.at[sl]], bn, sem))

    pend = start(0)
    for j in range(RPW):
        for c in pend:
            c.wait()
        if j + 1 < RPW:
            nxt = start(j + 1)
        bu, bp, bn = bufs[j % 2]

        def group(g, carry):
            for rr in range(LANES):
                r = g * LANES + rr
                off = pl.ds(j * COLS + r, 1)
                uo = ou_v[off][0]
                po = op_v[off][0]
                no = on_v[off][0]
                acc = None
                for c4 in range(F // LANES):
                    su = pl.ds(uo + c4 * LANES, LANES)
                    sp = pl.ds(po + c4 * LANES, LANES)
                    sn = pl.ds(no + c4 * LANES, LANES)
                    prod = bu[r, su] * (bp[r, sp] - bn[r, sn])
                    acc = prod if acc is None else acc + prod
                accs[2 * g + rr // 8, pl.ds((rr % 8) * LANES, LANES)] = acc
            return carry

        lax.fori_loop(0, COLS // LANES, group, 0)
        pltpu.sync_copy(accs, out_r.at[pl.ds(wid * (BPW // 8) + j * (COLS // 8),
                                             COLS // 8)])
        if j + 1 < RPW:
            pend = nxt


def _tc_body(a_ref, o_ref):
    x = a_ref[...]
    tot = jnp.zeros((), jnp.float32)
    for k in range(8):
        d = jnp.sum(x[:, k * LANES:(k + 1) * LANES], axis=1, keepdims=True)
        y = -d
        tot += jnp.sum(jnp.maximum(y, 0.0) + jnp.log1p(jnp.exp(-jnp.abs(y))))
    o_ref[0, 0] = tot


@jax.jit
def kernel(user, pos_i, neg_j, users_weight, items_weight):
    t_u = _transpose_table(users_weight.T)
    t_i = _transpose_table(items_weight.T)

    mesh = plsc.VectorSubcoreMesh(core_axis_name="c", subcore_axis_name="s")
    sc = functools.partial(
        pl.kernel,
        mesh=mesh,
        out_type=jax.ShapeDtypeStruct((B // 8, GR), jnp.float32),
        scratch_types=(
            [pltpu.VMEM((BPW,), jnp.int32)] * 6
            + [pltpu.VMEM((COLS, GR), jnp.float32)] * 6
            + [pltpu.VMEM((COLS // 8, GR), jnp.float32),
               pltpu.SemaphoreType.DMA]
        ),
    )(_sc_body)
    accs = sc(user, pos_i, neg_j, t_u, t_i)

    loss = pl.pallas_call(
        _tc_body,
        out_specs=pl.BlockSpec(memory_space=pltpu.SMEM),
        out_shape=jax.ShapeDtypeStruct((1, 1), jnp.float32),
    )(accs)
    return loss[0, 0]


# speedup vs baseline: 1.6571x; 1.0043x over previous
import functools

import jax
import jax.numpy as jnp
from jax import lax
from jax.experimental import pallas as pl
from jax.experimental.pallas import tpu as pltpu
from jax.experimental.pallas import tpu_sc as plsc

F = 64
GR = 2 * F
LANES = 16
NCORES = 2
NSUB = 16
NW = NCORES * NSUB
B = 16384
N_ROWS = 1000000
HALF = N_ROWS // 2
COLS = 128
RPW = 4
BPW = RPW * COLS
CB = 4096
TGRID = (N_ROWS + CB - 1) // CB
GRAN = TGRID * (CB // 2)


def _tr_body(x_ref, o_ref):
    t = jnp.swapaxes(x_ref[...], 0, 1)
    o_ref[:, 0:F] = t[0:CB // 2]
    o_ref[:, F:GR] = t[CB // 2:CB]


def _transpose_table(tw):
    return pl.pallas_call(
        _tr_body,
        grid=(TGRID,),
        in_specs=[pl.BlockSpec((F, CB), lambda k: (0, k))],
        out_specs=pl.BlockSpec((CB // 2, GR), lambda k: (k, 0)),
        out_shape=jax.ShapeDtypeStruct((GRAN, GR), jnp.float32),
    )(tw)


def _sc_body(user_r, pos_r, neg_r, tu_r, ti_r, out_r,
             iu_v, ip_v, in_v, ou_v, op_v, on_v,
             u0, p0, n0, u1, p1, n1, accs, sem):
    wid = lax.axis_index("s") * NCORES + lax.axis_index("c")
    base = wid * BPW
    pltpu.sync_copy(user_r.at[pl.ds(base, BPW)], iu_v)
    pltpu.sync_copy(pos_r.at[pl.ds(base, BPW)], ip_v)
    pltpu.sync_copy(neg_r.at[pl.ds(base, BPW)], in_v)

    for iv, ov in ((iu_v, ou_v), (ip_v, op_v), (in_v, on_v)):
        for c in range(BPW // LANES):
            sl = pl.ds(c * LANES, LANES)
            v = iv[sl]
            blk = lax.shift_left(lax.shift_right_logical(v, 12), 11)
            iv[sl] = blk + (v & 2047)
            ov[sl] = (lax.shift_right_logical(v, 11) & 1) * F

    bufs = ((u0, p0, n0), (u1, p1, n1))

    def start(j):
        bu, bp, bn = bufs[j % 2]
        sl = pl.ds(j * COLS, COLS)
        return (pltpu.async_copy(tu_r.at[iu_v.at[sl]], bu, sem),
                pltpu.async_copy(ti_r.at[ip_v.at[sl]], bp, sem),
                pltpu.async_copy(ti_r.at[in_v.at[sl]], bn, sem))

    pend = start(0)
    for j in range(RPW):
        for c in pend:
            c.wait()
        if j + 1 < RPW:
            nxt = start(j + 1)
        bu, bp, bn = bufs[j % 2]

        def group(g, carry):
            for rr in range(LANES):
                r = g * LANES + rr
                off = pl.ds(j * COLS + r, 1)
                uo = ou_v[off][0]
                po = op_v[off][0]
                no = on_v[off][0]
                acc = None
                for c4 in range(F // LANES):
                    su = pl.ds(uo + c4 * LANES, LANES)
                    sp = pl.ds(po + c4 * LANES, LANES)
                    sn = pl.ds(no + c4 * LANES, LANES)
                    prod = bu[r, su] * (bp[r, sp] - bn[r, sn])
                    acc = prod if acc is None else acc + prod
                accs[2 * g + rr // 8, pl.ds((rr % 8) * LANES, LANES)] = acc
            return carry

        lax.fori_loop(0, COLS // LANES, group, 0)
        pltpu.sync_copy(accs, out_r.at[pl.ds(wid * (BPW // 8) + j * (COLS // 8),
                                             COLS // 8)])
        if j + 1 < RPW:
            pend = nxt


def _tc_body(a_ref, o_ref):
    x = a_ref[...]
    tot = jnp.zeros((), jnp.float32)
    for k in range(8):
        d = jnp.sum(x[:, k * LANES:(k + 1) * LANES], axis=1, keepdims=True)
        y = -d
        tot += jnp.sum(jnp.maximum(y, 0.0) + jnp.log1p(jnp.exp(-jnp.abs(y))))
    o_ref[0, 0] = tot


@jax.jit
def kernel(user, pos_i, neg_j, users_weight, items_weight):
    t_u = _transpose_table(users_weight.T)
    t_i = _transpose_table(items_weight.T)

    mesh = plsc.VectorSubcoreMesh(core_axis_name="c", subcore_axis_name="s")
    sc = functools.partial(
        pl.kernel,
        mesh=mesh,
        out_type=jax.ShapeDtypeStruct((B // 8, GR), jnp.float32),
        scratch_types=(
            [pltpu.VMEM((BPW,), jnp.int32)] * 6
            + [pltpu.VMEM((COLS, GR), jnp.float32)] * 6
            + [pltpu.VMEM((COLS // 8, GR), jnp.float32),
               pltpu.SemaphoreType.DMA]
        ),
    )(_sc_body)
    accs = sc(user, pos_i, neg_j, t_u, t_i)

    loss = pl.pallas_call(
        _tc_body,
        out_specs=pl.BlockSpec(memory_space=pltpu.SMEM),
        out_shape=jax.ShapeDtypeStruct((1, 1), jnp.float32),
    )(accs)
    return loss[0, 0]


# speedup vs baseline: 2.3246x; 1.4028x over previous
import functools

import jax
import jax.numpy as jnp
from jax import lax
from jax.experimental import pallas as pl
from jax.experimental.pallas import tpu as pltpu
from jax.experimental.pallas import tpu_sc as plsc

F = 64
GR = 2 * F
LANES = 16
NCORES = 2
NSUB = 16
NW = NCORES * NSUB
B = 16384
N_ROWS = 1000000
HALF = N_ROWS // 2
COLS = 128
RPW = 4
BPW = RPW * COLS
CB = 16384
TGRID = (N_ROWS + CB - 1) // CB
GRAN = TGRID * (CB // 2)


def _tr_body(x_ref, o_ref):
    t = jnp.swapaxes(x_ref[...], 0, 1)
    o_ref[:, 0:F] = t[0:CB // 2]
    o_ref[:, F:GR] = t[CB // 2:CB]


def _transpose_table(tw):
    return pl.pallas_call(
        _tr_body,
        grid=(TGRID,),
        in_specs=[pl.BlockSpec((F, CB), lambda k: (0, k))],
        out_specs=pl.BlockSpec((CB // 2, GR), lambda k: (k, 0)),
        out_shape=jax.ShapeDtypeStruct((GRAN, GR), jnp.float32),
    )(tw)


def _sc_body(user_r, pos_r, neg_r, tu_r, ti_r, out_r,
             iu_v, ip_v, in_v, ou_v, op_v, on_v,
             u0, p0, n0, u1, p1, n1, accs, sem):
    wid = lax.axis_index("s") * NCORES + lax.axis_index("c")
    base = wid * BPW
    pltpu.sync_copy(user_r.at[pl.ds(base, BPW)], iu_v)
    pltpu.sync_copy(pos_r.at[pl.ds(base, BPW)], ip_v)
    pltpu.sync_copy(neg_r.at[pl.ds(base, BPW)], in_v)

    sh = CB.bit_length() - 1
    for iv, ov in ((iu_v, ou_v), (ip_v, op_v), (in_v, on_v)):
        for c in range(BPW // LANES):
            sl = pl.ds(c * LANES, LANES)
            v = iv[sl]
            blk = lax.shift_left(lax.shift_right_logical(v, sh), sh - 1)
            iv[sl] = blk + (v & (CB // 2 - 1))
            ov[sl] = (lax.shift_right_logical(v, sh - 1) & 1) * F

    bufs = ((u0, p0, n0), (u1, p1, n1))

    def start(j):
        bu, bp, bn = bufs[j % 2]
        sl = pl.ds(j * COLS, COLS)
        return (pltpu.async_copy(tu_r.at[iu_v.at[sl]], bu, sem),
                pltpu.async_copy(ti_r.at[ip_v.at[sl]], bp, sem),
                pltpu.async_copy(ti_r.at[in_v.at[sl]], bn, sem))

    pend = start(0)
    for j in range(RPW):
        for c in pend:
            c.wait()
        if j + 1 < RPW:
            nxt = start(j + 1)
        bu, bp, bn = bufs[j % 2]

        def group(g, carry):
            for rr in range(LANES):
                r = g * LANES + rr
                off = pl.ds(j * COLS + r, 1)
                uo = ou_v[off][0]
                po = op_v[off][0]
                no = on_v[off][0]
                acc = None
                for c4 in range(F // LANES):
                    su = pl.ds(uo + c4 * LANES, LANES)
                    sp = pl.ds(po + c4 * LANES, LANES)
                    sn = pl.ds(no + c4 * LANES, LANES)
                    prod = bu[r, su] * (bp[r, sp] - bn[r, sn])
                    acc = prod if acc is None else acc + prod
                accs[2 * g + rr // 8, pl.ds((rr % 8) * LANES, LANES)] = acc
            return carry

        lax.fori_loop(0, COLS // LANES, group, 0)
        pltpu.sync_copy(accs, out_r.at[pl.ds(wid * (BPW // 8) + j * (COLS // 8),
                                             COLS // 8)])
        if j + 1 < RPW:
            pend = nxt


def _tc_body(a_ref, o_ref):
    x = a_ref[...]
    tot = jnp.zeros((), jnp.float32)
    for k in range(8):
        d = jnp.sum(x[:, k * LANES:(k + 1) * LANES], axis=1, keepdims=True)
        y = -d
        tot += jnp.sum(jnp.maximum(y, 0.0) + jnp.log1p(jnp.exp(-jnp.abs(y))))
    o_ref[0, 0] = tot


@jax.jit
def kernel(user, pos_i, neg_j, users_weight, items_weight):
    t_u = _transpose_table(users_weight.T)
    t_i = _transpose_table(items_weight.T)

    mesh = plsc.VectorSubcoreMesh(core_axis_name="c", subcore_axis_name="s")
    sc = functools.partial(
        pl.kernel,
        mesh=mesh,
        out_type=jax.ShapeDtypeStruct((B // 8, GR), jnp.float32),
        scratch_types=(
            [pltpu.VMEM((BPW,), jnp.int32)] * 6
            + [pltpu.VMEM((COLS, GR), jnp.float32)] * 6
            + [pltpu.VMEM((COLS // 8, GR), jnp.float32),
               pltpu.SemaphoreType.DMA]
        ),
    )(_sc_body)
    accs = sc(user, pos_i, neg_j, t_u, t_i)

    loss = pl.pallas_call(
        _tc_body,
        out_specs=pl.BlockSpec(memory_space=pltpu.SMEM),
        out_shape=jax.ShapeDtypeStruct((1, 1), jnp.float32),
    )(accs)
    return loss[0, 0]


# speedup vs baseline: 2.4681x; 1.0617x over previous
import functools

import jax
import jax.numpy as jnp
from jax import lax
from jax.experimental import pallas as pl
from jax.experimental.pallas import tpu as pltpu
from jax.experimental.pallas import tpu_sc as plsc

F = 64
GR = 2 * F
LANES = 16
NCORES = 2
NSUB = 16
NW = NCORES * NSUB
B = 16384
N_ROWS = 1000000
HALF = N_ROWS // 2
COLS = 128
RPW = 4
BPW = RPW * COLS
CB = 32768
TGRID = (N_ROWS + CB - 1) // CB
GRAN = TGRID * (CB // 2)


def _tr_body(x_ref, o_ref):
    t = jnp.swapaxes(x_ref[...], 0, 1)
    o_ref[:, 0:F] = t[0:CB // 2]
    o_ref[:, F:GR] = t[CB // 2:CB]


def _transpose_table(tw):
    return pl.pallas_call(
        _tr_body,
        grid=(TGRID,),
        in_specs=[pl.BlockSpec((F, CB), lambda k: (0, k))],
        out_specs=pl.BlockSpec((CB // 2, GR), lambda k: (k, 0)),
        out_shape=jax.ShapeDtypeStruct((GRAN, GR), jnp.float32),
    )(tw)


def _sc_body(user_r, pos_r, neg_r, tu_r, ti_r, out_r,
             iu_v, ip_v, in_v, ou_v, op_v, on_v,
             u0, p0, n0, u1, p1, n1, accs, sem):
    wid = lax.axis_index("s") * NCORES + lax.axis_index("c")
    base = wid * BPW
    pltpu.sync_copy(user_r.at[pl.ds(base, BPW)], iu_v)
    pltpu.sync_copy(pos_r.at[pl.ds(base, BPW)], ip_v)
    pltpu.sync_copy(neg_r.at[pl.ds(base, BPW)], in_v)

    sh = CB.bit_length() - 1
    for iv, ov in ((iu_v, ou_v), (ip_v, op_v), (in_v, on_v)):
        for c in range(BPW // LANES):
            sl = pl.ds(c * LANES, LANES)
            v = iv[sl]
            blk = lax.shift_left(lax.shift_right_logical(v, sh), sh - 1)
            iv[sl] = blk + (v & (CB // 2 - 1))
            ov[sl] = (lax.shift_right_logical(v, sh - 1) & 1) * F

    bufs = ((u0, p0, n0), (u1, p1, n1))

    def start(j):
        bu, bp, bn = bufs[j % 2]
        sl = pl.ds(j * COLS, COLS)
        return (pltpu.async_copy(tu_r.at[iu_v.at[sl]], bu, sem),
                pltpu.async_copy(ti_r.at[ip_v.at[sl]], bp, sem),
                pltpu.async_copy(ti_r.at[in_v.at[sl]], bn, sem))

    pend = start(0)
    for j in range(RPW):
        for c in pend:
            c.wait()
        if j + 1 < RPW:
            nxt = start(j + 1)
        bu, bp, bn = bufs[j % 2]

        def group(g, carry):
            for rr in range(LANES):
                r = g * LANES + rr
                off = pl.ds(j * COLS + r, 1)
                uo = ou_v[off][0]
                po = op_v[off][0]
                no = on_v[off][0]
                acc = None
                for c4 in range(F // LANES):
                    su = pl.ds(uo + c4 * LANES, LANES)
                    sp = pl.ds(po + c4 * LANES, LANES)
                    sn = pl.ds(no + c4 * LANES, LANES)
                    prod = bu[r, su] * (bp[r, sp] - bn[r, sn])
                    acc = prod if acc is None else acc + prod
                accs[2 * g + rr // 8, pl.ds((rr % 8) * LANES, LANES)] = acc
            return carry

        lax.fori_loop(0, COLS // LANES, group, 0)
        pltpu.sync_copy(accs, out_r.at[pl.ds(wid * (BPW // 8) + j * (COLS // 8),
                                             COLS // 8)])
        if j + 1 < RPW:
            pend = nxt


def _tc_body(a_ref, o_ref):
    x = a_ref[...]
    tot = jnp.zeros((), jnp.float32)
    for k in range(8):
        d = jnp.sum(x[:, k * LANES:(k + 1) * LANES], axis=1, keepdims=True)
        y = -d
        tot += jnp.sum(jnp.maximum(y, 0.0) + jnp.log1p(jnp.exp(-jnp.abs(y))))
    o_ref[0, 0] = tot


@jax.jit
def kernel(user, pos_i, neg_j, users_weight, items_weight):
    t_u = _transpose_table(users_weight.T)
    t_i = _transpose_table(items_weight.T)

    mesh = plsc.VectorSubcoreMesh(core_axis_name="c", subcore_axis_name="s")
    sc = functools.partial(
        pl.kernel,
        mesh=mesh,
        out_type=jax.ShapeDtypeStruct((B // 8, GR), jnp.float32),
        scratch_types=(
            [pltpu.VMEM((BPW,), jnp.int32)] * 6
            + [pltpu.VMEM((COLS, GR), jnp.float32)] * 6
            + [pltpu.VMEM((COLS // 8, GR), jnp.float32),
               pltpu.SemaphoreType.DMA]
        ),
    )(_sc_body)
    accs = sc(user, pos_i, neg_j, t_u, t_i)

    loss = pl.pallas_call(
        _tc_body,
        out_specs=pl.BlockSpec(memory_space=pltpu.SMEM),
        out_shape=jax.ShapeDtypeStruct((1, 1), jnp.float32),
    )(accs)
    return loss[0, 0]
